# baseline (speedup 1.0000x reference)
import functools

import jax
import jax.numpy as jnp
from jax import lax
from jax.experimental import pallas as pl
from jax.experimental.pallas import tpu as pltpu
from jax.experimental.pallas import tpu_sc as plsc

N = 10000
E = 160000
D = 256
R = 200
RH = R // 2
H = 128
NS = 16
CHUNK = 128
NCHUNK = 80
EPT = NCHUNK * CHUNK
PAD = EPT - E // NS
QCH = 16
NP_ = 10240
RPT = NP_ // NS
ZROWS = 128
CNT = 1 << 20
CPT = CNT // NS
DUMP = N * RH

_MESH = plsc.VectorSubcoreMesh(core_axis_name="c", subcore_axis_name="s")


def _sc_hist(dst3, rel3):
    @functools.partial(
        pl.kernel,
        out_type=jax.ShapeDtypeStruct((2 * CNT,), jnp.float32),
        mesh=_MESH,
        scratch_types=[
            pltpu.VMEM((QCH, CHUNK), jnp.int32),
            pltpu.VMEM((QCH, CHUNK), jnp.int32),
            pltpu.VMEM((QCH, CHUNK), jnp.int32),
            pltpu.VMEM((CHUNK,), jnp.float32),
            pltpu.VMEM((CPT // 4,), jnp.float32),
            pltpu.VMEM_SHARED((CNT,), jnp.float32),
            pltpu.SemaphoreType.DMA,
        ],
    )
    def k(dst_hbm, rel_hbm, out_hbm, dst_q, rel_q, fid_q, ones_v, zbuf_v,
          cnt_sh, sem):
        c = lax.axis_index("c")
        s = lax.axis_index("s")
        zv = jnp.zeros((16,), jnp.float32)
        ov = jnp.ones((16,), jnp.float32)

        def zfill(i, carry):
            zbuf_v[pl.ds(i * 16, 16)] = zv
            return carry

        lax.fori_loop(0, CPT // 4 // 16, zfill, 0)
        for t in range(8):
            ones_v[pl.ds(t * 16, 16)] = ov
        base = s * CPT
        for t in range(4):
            pltpu.sync_copy(zbuf_v, cnt_sh.at[pl.ds(base + t * (CPT // 4),
                                                    CPT // 4)])
        plsc.subcore_barrier()

        rbase = c * RH
        for q in range(NCHUNK // QCH):
            pltpu.sync_copy(dst_hbm.at[s].at[pl.ds(q * QCH, QCH)], dst_q)
            pltpu.sync_copy(rel_hbm.at[s].at[pl.ds(q * QCH, QCH)], rel_q)

            def fid(j, carry):
                for g in range(CHUNK // 16):
                    dv = dst_q[j, pl.ds(g * 16, 16)]
                    rv = rel_q[j, pl.ds(g * 16, 16)] - rbase
                    ok = (rv >= 0) & (rv < RH) & (dv < N)
                    f = dv * RH + rv
                    fid_q[j, pl.ds(g * 16, 16)] = jnp.where(ok, f, DUMP)
                return carry

            lax.fori_loop(0, QCH, fid, 0)

            def fire(j, carry):
                pltpu.async_copy(ones_v, cnt_sh.at[fid_q.at[j]], sem,
                                 add=True)
                return carry

            lax.fori_loop(0, QCH, fire, 0)

            def drain(j, carry):
                pltpu.make_async_copy(ones_v, cnt_sh.at[fid_q.at[j]],
                                      sem).wait()
                return carry

            lax.fori_loop(0, QCH, drain, 0)

        plsc.subcore_barrier()
        pltpu.sync_copy(cnt_sh.at[pl.ds(base, CPT)],
                        out_hbm.at[pl.ds(c * CNT + base, CPT)])

    return k(dst3, rel3)


def _sc_layer(h0, h1, ra0, ra1, src3, dst3):
    @functools.partial(
        pl.kernel,
        out_type=jax.ShapeDtypeStruct((2 * NP_, H), jnp.float32),
        mesh=_MESH,
        scratch_types=[
            pltpu.VMEM((QCH, CHUNK), jnp.int32),
            pltpu.VMEM((QCH, CHUNK), jnp.int32),
            pltpu.VMEM((CHUNK, H), jnp.float32),
            pltpu.VMEM((CHUNK, H), jnp.float32),
            pltpu.VMEM_SHARED((NP_, H), jnp.float32),
            pltpu.SemaphoreType.DMA,
            pltpu.SemaphoreType.DMA,
            pltpu.SemaphoreType.DMA,
            pltpu.SemaphoreType.DMA,
        ],
    )
    def k(h0_hbm, h1_hbm, ra0_hbm, ra1_hbm, src_hbm, dst_hbm, out_hbm,
          idx_q, dst_q, rows0, rows1, agg_sh, gs0, gs1, ss0, ss1):
        c = lax.axis_index("c")
        s = lax.axis_index("s")
        row0 = s * RPT

        def init(ra_hbm):
            for t in range(RPT // ZROWS):
                pltpu.sync_copy(ra_hbm.at[pl.ds(row0 + t * ZROWS, ZROWS)],
                                agg_sh.at[pl.ds(row0 + t * ZROWS, ZROWS)])

        def phase(tab_hbm):
            for q in range(NCHUNK // QCH):
                pltpu.sync_copy(src_hbm.at[s].at[pl.ds(q * QCH, QCH)], idx_q)
                pltpu.sync_copy(dst_hbm.at[s].at[pl.ds(q * QCH, QCH)], dst_q)
                pltpu.async_copy(tab_hbm.at[idx_q.at[0]], rows0, gs0)
                pltpu.async_copy(tab_hbm.at[idx_q.at[1]], rows1, gs1)

                def body(t, carry):
                    j = 2 * t
                    pltpu.make_async_copy(tab_hbm.at[idx_q.at[j]], rows0,
                                          gs0).wait()
                    pltpu.async_copy(rows0, agg_sh.at[dst_q.at[j]], ss0,
                                     add=True)
                    pltpu.make_async_copy(tab_hbm.at[idx_q.at[j + 1]], rows1,
                                          gs1).wait()
                    pltpu.async_copy(rows1, agg_sh.at[dst_q.at[j + 1]], ss1,
                                     add=True)
                    pltpu.make_async_copy(rows0, agg_sh.at[dst_q.at[j]],
                                          ss0).wait()

                    @pl.when(j + 2 < QCH)
                    def _():
                        pltpu.async_copy(tab_hbm.at[idx_q.at[j + 2]], rows0,
                                         gs0)

                    pltpu.make_async_copy(rows1, agg_sh.at[dst_q.at[j + 1]],
                                          ss1).wait()

                    @pl.when(j + 3 < QCH)
                    def _():
                        pltpu.async_copy(tab_hbm.at[idx_q.at[j + 3]], rows1,
                                         gs1)
                    return carry

                lax.fori_loop(0, QCH // 2, body, 0)

        @pl.when(c == 0)
        def _():
            init(ra0_hbm)

        @pl.when(c == 1)
        def _():
            init(ra1_hbm)

        plsc.subcore_barrier()

        @pl.when(c == 0)
        def _():
            phase(h0_hbm)

        @pl.when(c == 1)
        def _():
            phase(h1_hbm)

        plsc.subcore_barrier()

        pltpu.sync_copy(agg_sh.at[pl.ds(row0, RPT)],
                        out_hbm.at[pl.ds(c * NP_ + row0, RPT)])

    return k(h0, h1, ra0, ra1, src3, dst3)


_BM = 1000


def _tc_relagg(cnt0, cnt1, rel_embed):
    grid = (N // _BM,)
    in_specs = [
        pl.BlockSpec((_BM, RH), lambda i: (i, 0)),
        pl.BlockSpec((_BM, RH), lambda i: (i, 0)),
        pl.BlockSpec((R, D), lambda i: (0, 0)),
    ]
    out_shape = (jax.ShapeDtypeStruct((NP_, H), jnp.float32),
                 jax.ShapeDtypeStruct((NP_, H), jnp.float32))
    out_specs = (pl.BlockSpec((_BM, H), lambda i: (i, 0)),
                 pl.BlockSpec((_BM, H), lambda i: (i, 0)))

    def body(c0_ref, c1_ref, re_ref, o0_ref, o1_ref):
        re = re_ref[...]
        ra = lax.dot_general(c0_ref[...], re[:RH, :], (((1,), (0,)), ((), ())),
                             preferred_element_type=jnp.float32,
                             precision=lax.Precision.HIGHEST)
        ra += lax.dot_general(c1_ref[...], re[RH:, :], (((1,), (0,)), ((), ())),
                              preferred_element_type=jnp.float32,
                              precision=lax.Precision.HIGHEST)
        o0_ref[...] = ra[:, :H]
        o1_ref[...] = ra[:, H:]

    return pl.pallas_call(body, grid=grid, in_specs=in_specs,
                          out_specs=out_specs,
                          out_shape=out_shape)(cnt0, cnt1, rel_embed)


def _tc_transform(S2, W, normv, split_out):
    grid = (N // _BM,)
    in_specs = [
        pl.BlockSpec((2, _BM, H), lambda i: (0, i, 0)),
        pl.BlockSpec((D, D), lambda i: (0, 0)),
        pl.BlockSpec((_BM, 1), lambda i: (i, 0)),
    ]
    if split_out:
        out_shape = (jax.ShapeDtypeStruct((N, H), jnp.float32),
                     jax.ShapeDtypeStruct((N, H), jnp.float32))
        out_specs = (pl.BlockSpec((_BM, H), lambda i: (i, 0)),
                     pl.BlockSpec((_BM, H), lambda i: (i, 0)))
    else:
        out_shape = jax.ShapeDtypeStruct((N, D), jnp.float32)
        out_specs = pl.BlockSpec((_BM, D), lambda i: (i, 0))

    def body(s_ref, w_ref, n_ref, *o_refs):
        s0 = s_ref[0]
        s1 = s_ref[1]
        w = w_ref[...]
        hA = lax.dot_general(s0, w[:, :H], (((1,), (1,)), ((), ())),
                             preferred_element_type=jnp.float32,
                             precision=lax.Precision.HIGHEST)
        hB = lax.dot_general(s1, w[:, H:], (((1,), (1,)), ((), ())),
                             preferred_element_type=jnp.float32,
                             precision=lax.Precision.HIGHEST)
        h = (hA + hB) * n_ref[...]
        if split_out:
            o_refs[0][...] = h[:, :H]
            o_refs[1][...] = h[:, H:]
        else:
            o_refs[0][...] = h

    return pl.pallas_call(body, grid=grid, in_specs=in_specs,
                          out_specs=out_specs, out_shape=out_shape)(S2, W, normv)


def kernel(ent_embed, rel_embed, norm, edge_index, ent_id, rel_id,
           W_r0, W_sl0, W_el0, W_r1, W_sl1, W_el1):
    def tile3(x, fill):
        x2 = x.reshape(NS, E // NS)
        return jnp.pad(x2, ((0, 0), (0, PAD)),
                       constant_values=fill).reshape(NS, NCHUNK, CHUNK)

    src3 = tile3(edge_index[0], 0)
    dst3 = tile3(edge_index[1], N)
    rel3 = tile3(rel_id, 0)
    h0 = ent_embed[:, :H]
    h1 = ent_embed[:, H:]

    cnt = _sc_hist(dst3, rel3)
    cnt0 = cnt[:N * RH].reshape(N, RH)
    cnt1 = cnt[CNT:CNT + N * RH].reshape(N, RH)
    ra0, ra1 = _tc_relagg(cnt0, cnt1, rel_embed)

    S0 = _sc_layer(h0, h1, ra0, ra1, src3, dst3).reshape(2, NP_, H)
    g0, g1 = _tc_transform(S0, W_r0, norm, split_out=True)
    S1 = _sc_layer(g0, g1, ra0, ra1, src3, dst3).reshape(2, NP_, H)
    return _tc_transform(S1, W_r1, norm, split_out=False)
